# baseline (speedup 1.0000x reference)
import functools

import jax
import jax.numpy as jnp
from jax import lax
from jax.experimental import pallas as pl
from jax.experimental.pallas import tpu as pltpu
from jax.experimental.pallas import tpu_sc as plsc

ENTITY_NUM = 1000000
RELATION_NUM = 1000
D = 64
B = 16384

NC = 2
NS = 16
NW = NC * NS
PER_W = B // NW
CH = 128
NCHUNK = PER_W // CH
G = 16


def _cs_tc_body(rel_ref, out_ref):
    r = rel_ref[...]
    out_ref[:, :D] = jnp.cos(r) + jnp.sin(r)
    out_ref[:, D:] = r


def _build_cr_table(relation_embedding):
    return pl.pallas_call(
        _cs_tc_body,
        out_shape=jax.ShapeDtypeStruct((RELATION_NUM, 2 * D), jnp.float32),
    )(relation_embedding)


def _sqrt16(x):
    xs = jnp.maximum(x, jnp.float32(1e-35))
    i = plsc.bitcast(xs, jnp.int32)
    i = jnp.int32(0x5F3759DF) - lax.shift_right_logical(i, jnp.int32(1))
    y = plsc.bitcast(i, jnp.float32)
    half = jnp.float32(0.5) * xs
    for _ in range(3):
        y = y * (jnp.float32(1.5) - half * y * y)
    return xs * y


def _compute_chunk(hb, tb, crb, obuf, c):
    def body(g, carry):
        rows = lax.iota(jnp.int32, G) + g * G
        acc = jnp.zeros((G,), jnp.float32)
        for d in range(D):
            dsp = jnp.full((G,), d, jnp.int32)
            dsp2 = jnp.full((G,), d + D, jnp.int32)
            h = plsc.load_gather(hb, [rows, dsp])
            t = plsc.load_gather(tb, [rows, dsp])
            cc = plsc.load_gather(crb, [rows, dsp])
            rr = plsc.load_gather(crb, [rows, dsp2])
            diff = h * cc + rr - t
            acc = acc + diff * diff
        off = pl.multiple_of(c * CH + g * G, G)
        obuf[pl.ds(off, G)] = _sqrt16(acc)
        return carry

    lax.fori_loop(0, CH // G, body, 0)


def _sc_body(ent_hbm, cr_hbm, head_hbm, tail_hbm, relidx_hbm, out_hbm,
             idxh, idxt, idxr, hbuf0, hbuf1, tbuf0, tbuf1, crbuf0, crbuf1,
             obuf, sem_idx, sem0, sem1):
    wid = lax.axis_index("s") * NC + lax.axis_index("c")
    base = wid * PER_W

    idx_copies = []
    for c in range(NCHUNK):
        off = pl.multiple_of(base + c * CH, CH)
        idx_copies.append(
            pltpu.async_copy(head_hbm.at[pl.ds(off, CH)], idxh.at[c], sem_idx))
        idx_copies.append(
            pltpu.async_copy(tail_hbm.at[pl.ds(off, CH)], idxt.at[c], sem_idx))
        idx_copies.append(
            pltpu.async_copy(relidx_hbm.at[pl.ds(off, CH)], idxr.at[c], sem_idx))
    for cp in idx_copies:
        cp.wait()

    hbufs = (hbuf0, hbuf1)
    tbufs = (tbuf0, tbuf1)
    crbufs = (crbuf0, crbuf1)
    sems = (sem0, sem1)

    def fire(c):
        slot = c % 2
        return [
            pltpu.async_copy(ent_hbm.at[idxh.at[c]], hbufs[slot], sems[slot]),
            pltpu.async_copy(ent_hbm.at[idxt.at[c]], tbufs[slot], sems[slot]),
            pltpu.async_copy(cr_hbm.at[idxr.at[c]], crbufs[slot], sems[slot]),
        ]

    pending = {0: fire(0)}
    for c in range(NCHUNK):
        if c + 1 < NCHUNK:
            pending[c + 1] = fire(c + 1)
        for cp in pending.pop(c):
            cp.wait()
        slot = c % 2
        _compute_chunk(hbufs[slot], tbufs[slot], crbufs[slot], obuf, c)

    pltpu.sync_copy(obuf, out_hbm.at[pl.ds(pl.multiple_of(base, PER_W), PER_W)])


_sc_score = functools.partial(
    pl.kernel,
    out_type=jax.ShapeDtypeStruct((B,), jnp.float32),
    compiler_params=pltpu.CompilerParams(
        needs_layout_passes=False, use_tc_tiling_on_sc=False),
    mesh=plsc.VectorSubcoreMesh(
        core_axis_name="c", subcore_axis_name="s", num_cores=NC,
        num_subcores=NS),
    scratch_types=[
        pltpu.VMEM((NCHUNK, CH), jnp.int32),
        pltpu.VMEM((NCHUNK, CH), jnp.int32),
        pltpu.VMEM((NCHUNK, CH), jnp.int32),
        pltpu.VMEM((CH, D), jnp.float32),
        pltpu.VMEM((CH, D), jnp.float32),
        pltpu.VMEM((CH, D), jnp.float32),
        pltpu.VMEM((CH, D), jnp.float32),
        pltpu.VMEM((CH, 2 * D), jnp.float32),
        pltpu.VMEM((CH, 2 * D), jnp.float32),
        pltpu.VMEM((PER_W,), jnp.float32),
        pltpu.SemaphoreType.DMA,
        pltpu.SemaphoreType.DMA,
        pltpu.SemaphoreType.DMA,
    ],
)(_sc_body)


def kernel(entity_embedding, relation_embedding, head, relation, tail):
    cr = _build_cr_table(relation_embedding)
    return _sc_score(entity_embedding, cr, head, tail, relation)


# speedup vs baseline: 1.5105x; 1.5105x over previous
import functools

import jax
import jax.numpy as jnp
from jax import lax
from jax.experimental import pallas as pl
from jax.experimental.pallas import tpu as pltpu
from jax.experimental.pallas import tpu_sc as plsc

ENTITY_NUM = 1000000
RELATION_NUM = 1000
D = 64
B = 16384
HT = 2 * B

NC = 2
NS = 16
NW = NC * NS

BLK = 512
NBLK = 1954
KMAX = 62
TAIL_BID = 1953
TAIL_LO = TAIL_BID * BLK
HITCAP = 8176
WROWS = 128
DUMP = HT
ROWS_N = HT + WROWS

_MESH = plsc.VectorSubcoreMesh(
    core_axis_name="c", subcore_axis_name="s", num_cores=NC, num_subcores=NS)
_PARAMS = pltpu.CompilerParams(
    needs_layout_passes=False, use_tc_tiling_on_sc=True)


def _cs_tc_body(rel_ref, out_ref):
    r = rel_ref[...]
    out_ref[:, :D] = jnp.cos(r) + jnp.sin(r)
    out_ref[:, D:] = r


def _build_cr_table(relation_embedding):
    return pl.pallas_call(
        _cs_tc_body,
        out_shape=jax.ShapeDtypeStruct((RELATION_NUM, 2 * D), jnp.float32),
    )(relation_embedding)


def _sqrt16(x):
    xs = jnp.maximum(x, jnp.float32(1e-35))
    i = plsc.bitcast(xs, jnp.int32)
    i = jnp.int32(0x5F3759DF) - lax.shift_right_logical(i, jnp.int32(1))
    y = plsc.bitcast(i, jnp.float32)
    half = jnp.float32(0.5) * xs
    for _ in range(3):
        y = y * (jnp.float32(1.5) - half * y * y)
    return xs * y


def _popcnt(m):
    return jnp.max(plsc.all_reduce_population_count(m))


_IOTA = None


def _phase1_body(ent_t, tailtab, ht_hbm, rows_hbm,
                 htbuf, hitE, hitP, blkE, blkP, stage, wbuf, wpos, sem):
    w = lax.axis_index("s") * NC + lax.axis_index("c")
    iota = lax.iota(jnp.int32, 16)

    pltpu.sync_copy(ht_hbm, htbuf)

    def passa(j, cnt):
        off = pl.multiple_of(j * 16, 16)
        e = htbuf[pl.ds(off, 16)]
        pos = iota + j * 16
        blkid = lax.shift_right_logical(e, jnp.int32(9))
        m = ((blkid - w) & jnp.int32(31)) == 0
        pc = _popcnt(m)

        def st(c):
            c2 = jnp.minimum(c, jnp.int32(HITCAP))
            plsc.store_compressed(hitE.at[pl.ds(c2, 16)], e, mask=m)
            plsc.store_compressed(hitP.at[pl.ds(c2, 16)], pos, mask=m)
            return c + pc

        return lax.cond(pc > 0, st, lambda c: c, cnt)

    cnt = lax.fori_loop(0, HT // 16, passa, jnp.int32(0))
    cnt = jnp.minimum(cnt, jnp.int32(HITCAP))

    for q in range(WROWS // 16):
        wpos[pl.ds(q * 16, 16)] = iota + jnp.int32(DUMP + q * 16)

    def flush(_wc):
        pltpu.async_copy(wbuf, rows_hbm.at[wpos], sem).wait()
        for q in range(WROWS // 16):
            wpos[pl.ds(q * 16, 16)] = iota + jnp.int32(DUMP + q * 16)
        return jnp.int32(0)

    nch = lax.shift_right_logical(cnt + 15, jnp.int32(4))

    def block(k, wcnt):
        bid = w + NW * k
        valid = bid <= TAIL_BID
        is_tail = bid == TAIL_BID
        blo = jnp.where(is_tail, jnp.int32(TAIL_LO), bid * BLK)
        bhi = blo + BLK

        def dma_tail(_):
            pltpu.sync_copy(tailtab, stage.at[:, pl.ds(0, 128)])
            return 0

        def dma_main(_):
            off = pl.multiple_of(bid * BLK, 128)
            pltpu.sync_copy(ent_t.at[:, pl.ds(off, BLK)], stage)
            return 0

        def do_block(wcnt):
            lax.cond(is_tail, dma_tail, dma_main, 0)

            def comp(ch, bcnt):
                off = pl.multiple_of(ch * 16, 16)
                e = hitE[pl.ds(off, 16)]
                p = hitP[pl.ds(off, 16)]
                lanes = iota + ch * 16
                m = (lanes < cnt) & (e >= blo) & (e < bhi)
                pc = _popcnt(m)

                def st(bc):
                    bc2 = jnp.minimum(bc, jnp.int32(HITCAP))
                    plsc.store_compressed(blkE.at[pl.ds(bc2, 16)], e, mask=m)
                    plsc.store_compressed(blkP.at[pl.ds(bc2, 16)], p, mask=m)
                    return bc + pc

                return lax.cond(pc > 0, st, lambda bc: bc, bcnt)

            bcnt = lax.fori_loop(0, nch, comp, jnp.int32(0))
            bcnt = jnp.minimum(bcnt, jnp.int32(HITCAP))
            ngr = lax.shift_right_logical(bcnt + 15, jnp.int32(4))

            def group(g, wc):
                wc = lax.cond(wc + 16 > WROWS, flush, lambda x: x, wc)
                off = pl.multiple_of(g * 16, 16)
                e = blkE[pl.ds(off, 16)]
                p = blkP[pl.ds(off, 16)]
                gm = (iota + g * 16) < bcnt
                eloc = jnp.clip(e - blo, 0, BLK - 1)
                rowv = wc + iota
                plsc.store_compressed(wpos.at[pl.ds(wc, 16)], p, mask=gm)
                for d in range(D):
                    dsp = jnp.full((16,), d, jnp.int32)
                    vals = plsc.load_gather(stage, [dsp, eloc], mask=gm)
                    plsc.store_scatter(wbuf, [rowv, dsp], vals, mask=gm)
                return wc + _popcnt(gm)

            return lax.fori_loop(0, ngr, group, wcnt)

        return lax.cond(valid, do_block, lambda x: x, wcnt)

    wcnt = lax.fori_loop(0, KMAX, block, jnp.int32(0))
    flush(wcnt)


_phase1 = functools.partial(
    pl.kernel,
    out_type=jax.ShapeDtypeStruct((ROWS_N, 2 * D), jnp.float32),
    compiler_params=_PARAMS,
    mesh=_MESH,
    scratch_types=[
        pltpu.VMEM((HT,), jnp.int32),
        pltpu.VMEM((HITCAP + 16,), jnp.int32),
        pltpu.VMEM((HITCAP + 16,), jnp.int32),
        pltpu.VMEM((HITCAP + 16,), jnp.int32),
        pltpu.VMEM((HITCAP + 16,), jnp.int32),
        pltpu.VMEM((D, BLK), jnp.float32),
        pltpu.VMEM((WROWS, 2 * D), jnp.float32),
        pltpu.VMEM((WROWS,), jnp.int32),
        pltpu.SemaphoreType.DMA,
    ],
)(_phase1_body)


CH = 128
PER_W = B // NW
NCHUNK = PER_W // CH


def _phase2_body(rows_hbm, cr_hbm, rel_hbm, out_hbm,
                 relbuf, hbuf, tbuf, crbuf, obuf, sem):
    w = lax.axis_index("s") * NC + lax.axis_index("c")
    base = w * PER_W
    iota = lax.iota(jnp.int32, 16)

    for c in range(NCHUNK):
        off = pl.multiple_of(base + c * CH, CH)
        pltpu.sync_copy(rel_hbm.at[pl.ds(off, CH)], relbuf.at[c])

    for c in range(NCHUNK):
        off = pl.multiple_of(base + c * CH, CH)
        cp1 = pltpu.async_copy(rows_hbm.at[pl.ds(off, CH), :], hbuf, sem)
        cp2 = pltpu.async_copy(
            rows_hbm.at[pl.ds(B + off, CH), :], tbuf, sem)
        cp3 = pltpu.async_copy(cr_hbm.at[relbuf.at[c]], crbuf, sem)
        cp1.wait()
        cp2.wait()
        cp3.wait()

        def group(g, carry):
            rows16 = iota + g * 16
            acc = [jnp.zeros((16,), jnp.float32) for _ in range(4)]
            for d in range(D):
                dsp = jnp.full((16,), d, jnp.int32)
                dsp2 = jnp.full((16,), d + D, jnp.int32)
                h = plsc.load_gather(hbuf, [rows16, dsp])
                t = plsc.load_gather(tbuf, [rows16, dsp])
                cc = plsc.load_gather(crbuf, [rows16, dsp])
                rr = plsc.load_gather(crbuf, [rows16, dsp2])
                diff = h * cc + rr - t
                acc[d % 4] = acc[d % 4] + diff * diff
            tot = (acc[0] + acc[1]) + (acc[2] + acc[3])
            off2 = pl.multiple_of(c * CH + g * 16, 16)
            obuf[pl.ds(off2, 16)] = _sqrt16(tot)
            return carry

        lax.fori_loop(0, CH // 16, group, 0)

    pltpu.sync_copy(obuf, out_hbm.at[pl.ds(pl.multiple_of(base, PER_W), PER_W)])


_phase2 = functools.partial(
    pl.kernel,
    out_type=jax.ShapeDtypeStruct((B,), jnp.float32),
    compiler_params=_PARAMS,
    mesh=_MESH,
    scratch_types=[
        pltpu.VMEM((NCHUNK, CH), jnp.int32),
        pltpu.VMEM((CH, 2 * D), jnp.float32),
        pltpu.VMEM((CH, 2 * D), jnp.float32),
        pltpu.VMEM((CH, 2 * D), jnp.float32),
        pltpu.VMEM((PER_W,), jnp.float32),
        pltpu.SemaphoreType.DMA,
    ],
)(_phase2_body)


def kernel(entity_embedding, relation_embedding, head, relation, tail):
    cr = _build_cr_table(relation_embedding)
    ent_t = jnp.swapaxes(entity_embedding, 0, 1)
    tailtab = jnp.pad(
        jnp.swapaxes(entity_embedding[TAIL_LO:], 0, 1), ((0, 0), (0, 64)))
    ht = jnp.concatenate([head, tail])
    rows = _phase1(ent_t, tailtab, ht)
    return _phase2(rows, cr, relation)


# speedup vs baseline: 2.5068x; 1.6595x over previous
import functools

import jax
import jax.numpy as jnp
from jax import lax
from jax.experimental import pallas as pl
from jax.experimental.pallas import tpu as pltpu
from jax.experimental.pallas import tpu_sc as plsc

ENTITY_NUM = 1000000
RELATION_NUM = 1000
D = 64
B = 16384
HT = 2 * B

NC = 2
NS = 16
NW = NC * NS

BLK = 512
KMAX = 62
TAIL_BID = 1953
TAIL_LO = TAIL_BID * BLK
HITCAP = 4080
WROWS = 128
DUMP = HT
ROWS_N = HT + WROWS

_MESH = plsc.VectorSubcoreMesh(
    core_axis_name="c", subcore_axis_name="s", num_cores=NC, num_subcores=NS)
_PARAMS_TILED = pltpu.CompilerParams(
    needs_layout_passes=False, use_tc_tiling_on_sc=True)
_PARAMS_LINEAR = pltpu.CompilerParams(
    needs_layout_passes=False, use_tc_tiling_on_sc=False)


def _cs_tc_body(rel_ref, out_ref):
    r = rel_ref[...]
    out_ref[:, :D] = jnp.cos(r) + jnp.sin(r)
    out_ref[:, D:] = r


def _build_cr_table(relation_embedding):
    return pl.pallas_call(
        _cs_tc_body,
        out_shape=jax.ShapeDtypeStruct((RELATION_NUM, 2 * D), jnp.float32),
    )(relation_embedding)


def _sqrt16(x):
    xs = jnp.maximum(x, jnp.float32(1e-35))
    i = plsc.bitcast(xs, jnp.int32)
    i = jnp.int32(0x5F3759DF) - lax.shift_right_logical(i, jnp.int32(1))
    y = plsc.bitcast(i, jnp.float32)
    half = jnp.float32(0.5) * xs
    for _ in range(3):
        y = y * (jnp.float32(1.5) - half * y * y)
    return xs * y


def _pc(m):
    return plsc.all_reduce_population_count(m)[0]


def _phase1_body(ent_t, tailtab, ht_hbm, rows_hbm,
                 htbuf, hitE, hitP, blkE, blkP, stage0, stage1, wbuf, wpos,
                 semA, semB, semW):
    w = lax.axis_index("s") * NC + lax.axis_index("c")
    iota = lax.iota(jnp.int32, 16)

    def passa_sc(s, cnt):
        hoff = pl.multiple_of(s * 4096, 128)
        pltpu.sync_copy(ht_hbm.at[pl.ds(hoff, 4096)], htbuf)

        def passa(j, cnt):
            off = pl.multiple_of(j * 16, 16)
            e = htbuf[pl.ds(off, 16)]
            pos = iota + s * 4096 + j * 16
            blkid = lax.shift_right_logical(e, jnp.int32(9))
            m = ((blkid - w) & jnp.int32(31)) == 0
            c2 = jnp.minimum(cnt, jnp.int32(HITCAP))
            plsc.store_compressed(hitE.at[pl.ds(c2, 16)], e, mask=m)
            plsc.store_compressed(hitP.at[pl.ds(c2, 16)], pos, mask=m)
            return cnt + _pc(m)

        return lax.fori_loop(0, 256, passa, cnt)

    cnt = lax.fori_loop(0, HT // 4096, passa_sc, jnp.int32(0))
    cnt = jnp.minimum(cnt, jnp.int32(HITCAP))
    nch = lax.shift_right_logical(cnt + 15, jnp.int32(4))

    for q in range(WROWS // 16):
        wpos[pl.ds(q * 16, 16)] = iota + jnp.int32(DUMP + q * 16)

    def flush(_wc):
        pltpu.async_copy(wbuf, rows_hbm.at[wpos], semW).wait()
        for q in range(WROWS // 16):
            wpos[pl.ds(q * 16, 16)] = iota + jnp.int32(DUMP + q * 16)
        return jnp.int32(0)

    def start_dma(k, stage, sem):
        bid = w + NW * k
        safe = jnp.where(bid <= TAIL_BID, bid, 0)
        is_tail = safe == TAIL_BID

        def tail_dma(_):
            pltpu.async_copy(tailtab, stage, sem)
            return 0

        def main_dma(_):
            off = pl.multiple_of(
                jnp.where(is_tail, 0, safe) * BLK, 128)
            pltpu.async_copy(ent_t.at[:, pl.ds(off, BLK)], stage, sem)
            return 0

        lax.cond(is_tail, tail_dma, main_dma, 0)

    def process(k, stage, sem, wcnt):
        bid = w + NW * k
        valid = bid <= TAIL_BID
        is_tail = bid == TAIL_BID
        blo = jnp.where(is_tail, jnp.int32(TAIL_LO), bid * BLK)
        bhi = blo + BLK
        pltpu.make_async_copy(ent_t.at[:, pl.ds(0, BLK)], stage, sem).wait()

        def do_block(wcnt):
            def comp(ch, bcnt):
                off = pl.multiple_of(ch * 16, 16)
                e = hitE[pl.ds(off, 16)]
                p = hitP[pl.ds(off, 16)]
                lanes = iota + ch * 16
                m = (lanes < cnt) & (e >= blo) & (e < bhi)
                bc2 = jnp.minimum(bcnt, jnp.int32(HITCAP))
                plsc.store_compressed(blkE.at[pl.ds(bc2, 16)], e, mask=m)
                plsc.store_compressed(blkP.at[pl.ds(bc2, 16)], p, mask=m)
                return bcnt + _pc(m)

            bcnt = lax.fori_loop(0, nch, comp, jnp.int32(0))
            bcnt = jnp.minimum(bcnt, jnp.int32(HITCAP))
            ngr = lax.shift_right_logical(bcnt + 15, jnp.int32(4))

            def group(g, wc):
                wc = lax.cond(wc + 16 > WROWS, flush, lambda x: x, wc)
                off = pl.multiple_of(g * 16, 16)
                e = blkE[pl.ds(off, 16)]
                p = blkP[pl.ds(off, 16)]
                gm = (iota + g * 16) < bcnt
                eloc = jnp.clip(e - blo, 0, BLK - 1)
                rowv = wc + iota
                plsc.store_compressed(wpos.at[pl.ds(wc, 16)], p, mask=gm)
                for d in range(D):
                    dsp = jnp.full((16,), d, jnp.int32)
                    vals = plsc.load_gather(stage, [dsp, eloc], mask=gm)
                    plsc.store_scatter(wbuf, [rowv, dsp], vals, mask=gm)
                return wc + _pc(gm)

            return lax.fori_loop(0, ngr, group, wcnt)

        return lax.cond(valid, do_block, lambda x: x, wcnt)

    start_dma(jnp.int32(0), stage0, semA)

    def pair(k2, wcnt):
        ka = 2 * k2
        start_dma(ka + 1, stage1, semB)
        wcnt = process(ka, stage0, semA, wcnt)
        start_dma(ka + 2, stage0, semA)
        wcnt = process(ka + 1, stage1, semB, wcnt)
        return wcnt

    wcnt = lax.fori_loop(0, KMAX // 2, pair, jnp.int32(0))
    pltpu.make_async_copy(ent_t.at[:, pl.ds(0, BLK)], stage0, semA).wait()
    flush(wcnt)


_phase1 = functools.partial(
    pl.kernel,
    out_type=jax.ShapeDtypeStruct((ROWS_N, 2 * D), jnp.float32),
    compiler_params=_PARAMS_TILED,
    mesh=_MESH,
    scratch_types=[
        pltpu.VMEM((4096,), jnp.int32),
        pltpu.VMEM((HITCAP + 16,), jnp.int32),
        pltpu.VMEM((HITCAP + 16,), jnp.int32),
        pltpu.VMEM((HITCAP + 16,), jnp.int32),
        pltpu.VMEM((HITCAP + 16,), jnp.int32),
        pltpu.VMEM((D, BLK), jnp.float32),
        pltpu.VMEM((D, BLK), jnp.float32),
        pltpu.VMEM((WROWS, 2 * D), jnp.float32),
        pltpu.VMEM((WROWS,), jnp.int32),
        pltpu.SemaphoreType.DMA,
        pltpu.SemaphoreType.DMA,
        pltpu.SemaphoreType.DMA,
    ],
)(_phase1_body)


CH = 128
PER_W = B // NW
NCHUNK = PER_W // CH


def _phase2_body(rows_hbm, cr_hbm, rel_hbm, out_hbm,
                 relbuf, hbuf, tbuf, crbuf, obuf, sem):
    w = lax.axis_index("s") * NC + lax.axis_index("c")
    base = w * PER_W
    iota = lax.iota(jnp.int32, 16)

    for c in range(NCHUNK):
        off = pl.multiple_of(base + c * CH, CH)
        pltpu.sync_copy(rel_hbm.at[pl.ds(off, CH)], relbuf.at[c])

    for c in range(NCHUNK):
        off = pl.multiple_of(base + c * CH, CH)
        cp1 = pltpu.async_copy(rows_hbm.at[pl.ds(off, CH), :], hbuf, sem)
        cp2 = pltpu.async_copy(
            rows_hbm.at[pl.ds(B + off, CH), :], tbuf, sem)
        cp3 = pltpu.async_copy(cr_hbm.at[relbuf.at[c]], crbuf, sem)
        cp1.wait()
        cp2.wait()
        cp3.wait()

        def group(g, carry):
            rows16 = iota + g * 16
            acc = [jnp.zeros((16,), jnp.float32) for _ in range(4)]
            for d in range(D):
                dsp = jnp.full((16,), d, jnp.int32)
                dsp2 = jnp.full((16,), d + D, jnp.int32)
                h = plsc.load_gather(hbuf, [rows16, dsp])
                t = plsc.load_gather(tbuf, [rows16, dsp])
                cc = plsc.load_gather(crbuf, [rows16, dsp])
                rr = plsc.load_gather(crbuf, [rows16, dsp2])
                diff = h * cc + rr - t
                acc[d % 4] = acc[d % 4] + diff * diff
            tot = (acc[0] + acc[1]) + (acc[2] + acc[3])
            off2 = pl.multiple_of(c * CH + g * 16, 16)
            obuf[pl.ds(off2, 16)] = _sqrt16(tot)
            return carry

        lax.fori_loop(0, CH // 16, group, 0)

    pltpu.sync_copy(obuf, out_hbm.at[pl.ds(pl.multiple_of(base, PER_W), PER_W)])


_phase2 = functools.partial(
    pl.kernel,
    out_type=jax.ShapeDtypeStruct((B,), jnp.float32),
    compiler_params=_PARAMS_LINEAR,
    mesh=_MESH,
    scratch_types=[
        pltpu.VMEM((NCHUNK, CH), jnp.int32),
        pltpu.VMEM((CH, 2 * D), jnp.float32),
        pltpu.VMEM((CH, 2 * D), jnp.float32),
        pltpu.VMEM((CH, 2 * D), jnp.float32),
        pltpu.VMEM((PER_W,), jnp.float32),
        pltpu.SemaphoreType.DMA,
    ],
)(_phase2_body)


def kernel(entity_embedding, relation_embedding, head, relation, tail):
    cr = _build_cr_table(relation_embedding)
    ent_t = jnp.swapaxes(entity_embedding, 0, 1)
    tailtab = jnp.pad(
        jnp.swapaxes(entity_embedding[TAIL_LO:], 0, 1), ((0, 0), (0, 448)))
    ht = jnp.concatenate([head, tail])
    rows = _phase1(ent_t, tailtab, ht)
    return _phase2(rows, cr, relation)


# speedup vs baseline: 3.2573x; 1.2994x over previous
import functools

import jax
import jax.numpy as jnp
from jax import lax
from jax.experimental import pallas as pl
from jax.experimental.pallas import tpu as pltpu
from jax.experimental.pallas import tpu_sc as plsc

ENTITY_NUM = 1000000
RELATION_NUM = 1000
D = 64
B = 16384
HT = 2 * B

NC = 2
NS = 16
NW = NC * NS

BLK = 512
KMAX = 62
TAIL_BID = 1953
TAIL_LO = TAIL_BID * BLK
HITCAP = 4080
WROWS = 128
DUMP = HT
ROWS_N = HT + WROWS

_MESH = plsc.VectorSubcoreMesh(
    core_axis_name="c", subcore_axis_name="s", num_cores=NC, num_subcores=NS)
_PARAMS_TILED = pltpu.CompilerParams(
    needs_layout_passes=False, use_tc_tiling_on_sc=True)
_PARAMS_LINEAR = pltpu.CompilerParams(
    needs_layout_passes=False, use_tc_tiling_on_sc=False)


def _cs_tc_body(rel_ref, out_ref):
    r = rel_ref[...]
    out_ref[:, :D] = jnp.cos(r) + jnp.sin(r)
    out_ref[:, D:] = r


def _build_cr_table(relation_embedding):
    return pl.pallas_call(
        _cs_tc_body,
        out_shape=jax.ShapeDtypeStruct((RELATION_NUM, 2 * D), jnp.float32),
    )(relation_embedding)


def _sqrt16(x):
    xs = jnp.maximum(x, jnp.float32(1e-35))
    i = plsc.bitcast(xs, jnp.int32)
    i = jnp.int32(0x5F3759DF) - lax.shift_right_logical(i, jnp.int32(1))
    y = plsc.bitcast(i, jnp.float32)
    half = jnp.float32(0.5) * xs
    for _ in range(3):
        y = y * (jnp.float32(1.5) - half * y * y)
    return xs * y


def _pc(m):
    return plsc.all_reduce_population_count(m)[0]


def _phase1_body(ent_t, tailtab, ht_hbm, rows_hbm,
                 htbuf, hitE, hitP, blkE, blkP, stage0, stage1, wbuf, wpos,
                 semA, semB, semW):
    w = lax.axis_index("s") * NC + lax.axis_index("c")
    iota = lax.iota(jnp.int32, 16)

    def passa_sc(s, cnt):
        hoff = pl.multiple_of(s * 4096, 128)
        pltpu.sync_copy(ht_hbm.at[pl.ds(hoff, 4096)], htbuf)

        def passa(j, cnt):
            off = pl.multiple_of(j * 16, 16)
            e = htbuf[pl.ds(off, 16)]
            pos = iota + s * 4096 + j * 16
            blkid = lax.shift_right_logical(e, jnp.int32(9))
            m = ((blkid - w) & jnp.int32(31)) == 0
            c2 = jnp.minimum(cnt, jnp.int32(HITCAP))
            plsc.store_compressed(hitE.at[pl.ds(c2, 16)], e, mask=m)
            plsc.store_compressed(hitP.at[pl.ds(c2, 16)], pos, mask=m)
            return cnt + _pc(m)

        return lax.fori_loop(0, 256, passa, cnt)

    cnt = lax.fori_loop(0, HT // 4096, passa_sc, jnp.int32(0))
    cnt = jnp.minimum(cnt, jnp.int32(HITCAP))
    nch = lax.shift_right_logical(cnt + 15, jnp.int32(4))

    for q in range(WROWS // 16):
        wpos[pl.ds(q * 16, 16)] = iota + jnp.int32(DUMP + q * 16)

    def flush(_wc):
        pltpu.async_copy(wbuf, rows_hbm.at[wpos], semW).wait()
        for q in range(WROWS // 16):
            wpos[pl.ds(q * 16, 16)] = iota + jnp.int32(DUMP + q * 16)
        return jnp.int32(0)

    def start_dma(k, stage, sem):
        bid = w + NW * k
        safe = jnp.where(bid <= TAIL_BID, bid, 0)
        is_tail = safe == TAIL_BID

        def tail_dma(_):
            pltpu.async_copy(tailtab, stage, sem)
            return 0

        def main_dma(_):
            off = pl.multiple_of(
                jnp.where(is_tail, 0, safe) * BLK, 128)
            pltpu.async_copy(ent_t.at[:, pl.ds(off, BLK)], stage, sem)
            return 0

        lax.cond(is_tail, tail_dma, main_dma, 0)

    def process(k, stage, sem, wcnt):
        bid = w + NW * k
        valid = bid <= TAIL_BID
        is_tail = bid == TAIL_BID
        blo = jnp.where(is_tail, jnp.int32(TAIL_LO), bid * BLK)
        bhi = blo + BLK
        pltpu.make_async_copy(ent_t.at[:, pl.ds(0, BLK)], stage, sem).wait()

        def do_block(wcnt):
            def comp(ch, bcnt):
                off = pl.multiple_of(ch * 16, 16)
                e = hitE[pl.ds(off, 16)]
                p = hitP[pl.ds(off, 16)]
                lanes = iota + ch * 16
                m = (lanes < cnt) & (e >= blo) & (e < bhi)
                bc2 = jnp.minimum(bcnt, jnp.int32(HITCAP))
                plsc.store_compressed(blkE.at[pl.ds(bc2, 16)], e, mask=m)
                plsc.store_compressed(blkP.at[pl.ds(bc2, 16)], p, mask=m)
                return bcnt + _pc(m)

            bcnt = lax.fori_loop(0, nch, comp, jnp.int32(0))
            bcnt = jnp.minimum(bcnt, jnp.int32(HITCAP))
            ngr = lax.shift_right_logical(bcnt + 15, jnp.int32(4))

            def group(g, wc):
                wc = lax.cond(wc + 16 > WROWS, flush, lambda x: x, wc)
                off = pl.multiple_of(g * 16, 16)
                e = blkE[pl.ds(off, 16)]
                p = blkP[pl.ds(off, 16)]
                gm = (iota + g * 16) < bcnt
                eloc = jnp.clip(e - blo, 0, BLK - 1)
                rowv = wc + iota
                plsc.store_compressed(wpos.at[pl.ds(wc, 16)], p, mask=gm)
                for d in range(D):
                    dsp = jnp.full((16,), d, jnp.int32)
                    vals = plsc.load_gather(stage, [dsp, eloc], mask=gm)
                    plsc.store_scatter(wbuf, [rowv, dsp], vals, mask=gm)
                return wc + _pc(gm)

            return lax.fori_loop(0, ngr, group, wcnt)

        return lax.cond(valid, do_block, lambda x: x, wcnt)

    start_dma(jnp.int32(0), stage0, semA)

    def pair(k2, wcnt):
        ka = 2 * k2
        start_dma(ka + 1, stage1, semB)
        wcnt = process(ka, stage0, semA, wcnt)
        start_dma(ka + 2, stage0, semA)
        wcnt = process(ka + 1, stage1, semB, wcnt)
        return wcnt

    wcnt = lax.fori_loop(0, KMAX // 2, pair, jnp.int32(0))
    pltpu.make_async_copy(ent_t.at[:, pl.ds(0, BLK)], stage0, semA).wait()
    flush(wcnt)


_phase1 = functools.partial(
    pl.kernel,
    out_type=jax.ShapeDtypeStruct((ROWS_N, 2 * D), jnp.float32),
    compiler_params=_PARAMS_TILED,
    mesh=_MESH,
    scratch_types=[
        pltpu.VMEM((4096,), jnp.int32),
        pltpu.VMEM((HITCAP + 16,), jnp.int32),
        pltpu.VMEM((HITCAP + 16,), jnp.int32),
        pltpu.VMEM((HITCAP + 16,), jnp.int32),
        pltpu.VMEM((HITCAP + 16,), jnp.int32),
        pltpu.VMEM((D, BLK), jnp.float32),
        pltpu.VMEM((D, BLK), jnp.float32),
        pltpu.VMEM((WROWS, 2 * D), jnp.float32),
        pltpu.VMEM((WROWS,), jnp.int32),
        pltpu.SemaphoreType.DMA,
        pltpu.SemaphoreType.DMA,
        pltpu.SemaphoreType.DMA,
    ],
)(_phase1_body)


CH = 128
PER_W = B // NW
NCHUNK = PER_W // CH


def _phase2_body(rows_hbm, cr_hbm, rel_hbm, out_hbm,
                 relbuf, hbuf, tbuf, crbuf, obuf, sem):
    w = lax.axis_index("s") * NC + lax.axis_index("c")
    base = w * PER_W
    iota = lax.iota(jnp.int32, 16)

    for c in range(NCHUNK):
        off = pl.multiple_of(base + c * CH, CH)
        pltpu.sync_copy(rel_hbm.at[pl.ds(off, CH)], relbuf.at[c])

    for c in range(NCHUNK):
        off = pl.multiple_of(base + c * CH, CH)
        cp1 = pltpu.async_copy(rows_hbm.at[pl.ds(off, CH), :], hbuf, sem)
        cp2 = pltpu.async_copy(
            rows_hbm.at[pl.ds(B + off, CH), :], tbuf, sem)
        cp3 = pltpu.async_copy(cr_hbm.at[relbuf.at[c]], crbuf, sem)
        cp1.wait()
        cp2.wait()
        cp3.wait()

        def row(i, carry):
            s = jnp.zeros((16,), jnp.float32)
            for q in range(D // 16):
                qo = pl.multiple_of(q * 16, 16)
                qo2 = pl.multiple_of(D + q * 16, 16)
                h = hbuf[i, pl.ds(qo, 16)]
                t = tbuf[i, pl.ds(qo, 16)]
                cc = crbuf[i, pl.ds(qo, 16)]
                rr = crbuf[i, pl.ds(qo2, 16)]
                diff = h * cc + rr - t
                s = s + diff * diff
            for sh in (1, 2, 4, 8):
                s = s + s.at[iota ^ sh].get(mode="promise_in_bounds")
            lane = i & jnp.int32(15)
            m = iota == lane
            idx16 = jnp.full((16,), c * CH, jnp.int32) + i
            plsc.store_scatter(obuf, [idx16], _sqrt16(s), mask=m)
            return carry

        lax.fori_loop(0, CH, row, 0)

    pltpu.sync_copy(obuf, out_hbm.at[pl.ds(pl.multiple_of(base, PER_W), PER_W)])


_phase2 = functools.partial(
    pl.kernel,
    out_type=jax.ShapeDtypeStruct((B,), jnp.float32),
    compiler_params=_PARAMS_LINEAR,
    mesh=_MESH,
    scratch_types=[
        pltpu.VMEM((NCHUNK, CH), jnp.int32),
        pltpu.VMEM((CH, 2 * D), jnp.float32),
        pltpu.VMEM((CH, 2 * D), jnp.float32),
        pltpu.VMEM((CH, 2 * D), jnp.float32),
        pltpu.VMEM((PER_W,), jnp.float32),
        pltpu.SemaphoreType.DMA,
    ],
)(_phase2_body)


def kernel(entity_embedding, relation_embedding, head, relation, tail):
    cr = _build_cr_table(relation_embedding)
    ent_t = jnp.swapaxes(entity_embedding, 0, 1)
    tailtab = jnp.pad(
        jnp.swapaxes(entity_embedding[TAIL_LO:], 0, 1), ((0, 0), (0, 448)))
    ht = jnp.concatenate([head, tail])
    rows = _phase1(ent_t, tailtab, ht)
    return _phase2(rows, cr, relation)


# speedup vs baseline: 3.3898x; 1.0407x over previous
import functools

import jax
import jax.numpy as jnp
from jax import lax
from jax.experimental import pallas as pl
from jax.experimental.pallas import tpu as pltpu
from jax.experimental.pallas import tpu_sc as plsc

ENTITY_NUM = 1000000
RELATION_NUM = 1000
D = 64
B = 16384
HT = 2 * B

NC = 2
NS = 16
NW = NC * NS

BLK = 512
KMAX = 62
TAIL_BID = 1953
TAIL_LO = TAIL_BID * BLK
HITCAP = 4080
WROWS = 128
DUMP = HT
ROWS_N = HT + WROWS

_MESH = plsc.VectorSubcoreMesh(
    core_axis_name="c", subcore_axis_name="s", num_cores=NC, num_subcores=NS)
_PARAMS_TILED = pltpu.CompilerParams(
    needs_layout_passes=False, use_tc_tiling_on_sc=True)
_PARAMS_LINEAR = pltpu.CompilerParams(
    needs_layout_passes=False, use_tc_tiling_on_sc=False)


def _cs_tc_body(rel_ref, out_ref):
    r = rel_ref[...]
    out_ref[:, :D] = jnp.cos(r) + jnp.sin(r)
    out_ref[:, D:] = r


def _build_cr_table(relation_embedding):
    return pl.pallas_call(
        _cs_tc_body,
        out_shape=jax.ShapeDtypeStruct((RELATION_NUM, 2 * D), jnp.float32),
    )(relation_embedding)


def _sqrt16(x):
    xs = jnp.maximum(x, jnp.float32(1e-35))
    i = plsc.bitcast(xs, jnp.int32)
    i = jnp.int32(0x5F3759DF) - lax.shift_right_logical(i, jnp.int32(1))
    y = plsc.bitcast(i, jnp.float32)
    half = jnp.float32(0.5) * xs
    for _ in range(3):
        y = y * (jnp.float32(1.5) - half * y * y)
    return xs * y


def _pc(m):
    return plsc.all_reduce_population_count(m)[0]


def _phase1_body(ent_t, tailtab, ht_hbm, rows_hbm,
                 htbuf, hitE, hitP, blkE, blkP, stage0, stage1, wbuf, wpos,
                 semA, semB, semW):
    w = lax.axis_index("s") * NC + lax.axis_index("c")
    iota = lax.iota(jnp.int32, 16)

    pltpu.async_copy(
        ent_t.at[:, pl.ds(pl.multiple_of(w * BLK, 128), BLK)], stage0, semA)

    def passa_sc(s, cnt):
        hoff = pl.multiple_of(s * 4096, 128)
        pltpu.sync_copy(ht_hbm.at[pl.ds(hoff, 4096)], htbuf)

        def passa(j, cnt):
            off = pl.multiple_of(j * 16, 16)
            e = htbuf[pl.ds(off, 16)]
            pos = iota + s * 4096 + j * 16
            blkid = lax.shift_right_logical(e, jnp.int32(9))
            m = ((blkid - w) & jnp.int32(31)) == 0
            c2 = jnp.minimum(cnt, jnp.int32(HITCAP))
            plsc.store_compressed(hitE.at[pl.ds(c2, 16)], e, mask=m)
            plsc.store_compressed(hitP.at[pl.ds(c2, 16)], pos, mask=m)
            return cnt + _pc(m)

        return lax.fori_loop(0, 256, passa, cnt)

    cnt = lax.fori_loop(0, HT // 4096, passa_sc, jnp.int32(0))
    cnt = jnp.minimum(cnt, jnp.int32(HITCAP))
    nch = lax.shift_right_logical(cnt + 15, jnp.int32(4))

    for q in range(WROWS // 16):
        wpos[pl.ds(q * 16, 16)] = iota + jnp.int32(DUMP + q * 16)

    def flush(_wc):
        pltpu.async_copy(wbuf, rows_hbm.at[wpos], semW).wait()
        for q in range(WROWS // 16):
            wpos[pl.ds(q * 16, 16)] = iota + jnp.int32(DUMP + q * 16)
        return jnp.int32(0)

    def start_dma(k, stage, sem):
        bid = w + NW * k
        safe = jnp.where(bid <= TAIL_BID, bid, 0)
        is_tail = safe == TAIL_BID

        def tail_dma(_):
            pltpu.async_copy(tailtab, stage, sem)
            return 0

        def main_dma(_):
            off = pl.multiple_of(
                jnp.where(is_tail, 0, safe) * BLK, 128)
            pltpu.async_copy(ent_t.at[:, pl.ds(off, BLK)], stage, sem)
            return 0

        lax.cond(is_tail, tail_dma, main_dma, 0)

    def process(k, stage, sem, wcnt):
        bid = w + NW * k
        valid = bid <= TAIL_BID
        is_tail = bid == TAIL_BID
        blo = jnp.where(is_tail, jnp.int32(TAIL_LO), bid * BLK)
        bhi = blo + BLK
        pltpu.make_async_copy(ent_t.at[:, pl.ds(0, BLK)], stage, sem).wait()

        def do_block(wcnt):
            def comp(ch, bcnt):
                off = pl.multiple_of(ch * 16, 16)
                e = hitE[pl.ds(off, 16)]
                p = hitP[pl.ds(off, 16)]
                lanes = iota + ch * 16
                m = (lanes < cnt) & (e >= blo) & (e < bhi)
                bc2 = jnp.minimum(bcnt, jnp.int32(HITCAP))
                plsc.store_compressed(blkE.at[pl.ds(bc2, 16)], e, mask=m)
                plsc.store_compressed(blkP.at[pl.ds(bc2, 16)], p, mask=m)
                return bcnt + _pc(m)

            bcnt = lax.fori_loop(0, nch, comp, jnp.int32(0))
            bcnt = jnp.minimum(bcnt, jnp.int32(HITCAP))
            ngr = lax.shift_right_logical(bcnt + 15, jnp.int32(4))

            def group(g, wc):
                wc = lax.cond(wc + 16 > WROWS, flush, lambda x: x, wc)
                off = pl.multiple_of(g * 16, 16)
                e = blkE[pl.ds(off, 16)]
                p = blkP[pl.ds(off, 16)]
                gm = (iota + g * 16) < bcnt
                eloc = jnp.clip(e - blo, 0, BLK - 1)
                rowv = wc + iota
                plsc.store_compressed(wpos.at[pl.ds(wc, 16)], p, mask=gm)
                for d in range(D):
                    dsp = jnp.full((16,), d, jnp.int32)
                    vals = plsc.load_gather(stage, [dsp, eloc], mask=gm)
                    plsc.store_scatter(wbuf, [rowv, dsp], vals, mask=gm)
                return wc + _pc(gm)

            return lax.fori_loop(0, ngr, group, wcnt)

        return lax.cond(valid, do_block, lambda x: x, wcnt)

    def pair(k2, wcnt):
        ka = 2 * k2
        start_dma(ka + 1, stage1, semB)
        wcnt = process(ka, stage0, semA, wcnt)
        start_dma(ka + 2, stage0, semA)
        wcnt = process(ka + 1, stage1, semB, wcnt)
        return wcnt

    wcnt = lax.fori_loop(0, KMAX // 2, pair, jnp.int32(0))
    pltpu.make_async_copy(ent_t.at[:, pl.ds(0, BLK)], stage0, semA).wait()
    flush(wcnt)


_phase1 = functools.partial(
    pl.kernel,
    out_type=jax.ShapeDtypeStruct((ROWS_N, 2 * D), jnp.float32),
    compiler_params=_PARAMS_TILED,
    mesh=_MESH,
    scratch_types=[
        pltpu.VMEM((4096,), jnp.int32),
        pltpu.VMEM((HITCAP + 16,), jnp.int32),
        pltpu.VMEM((HITCAP + 16,), jnp.int32),
        pltpu.VMEM((HITCAP + 16,), jnp.int32),
        pltpu.VMEM((HITCAP + 16,), jnp.int32),
        pltpu.VMEM((D, BLK), jnp.float32),
        pltpu.VMEM((D, BLK), jnp.float32),
        pltpu.VMEM((WROWS, 2 * D), jnp.float32),
        pltpu.VMEM((WROWS,), jnp.int32),
        pltpu.SemaphoreType.DMA,
        pltpu.SemaphoreType.DMA,
        pltpu.SemaphoreType.DMA,
    ],
)(_phase1_body)


CH = 128
PER_W = B // NW
NCHUNK = PER_W // CH


def _phase2_body(rows_hbm, cr_hbm, rel_hbm, out_hbm,
                 relbuf, hbuf, hbuf2, tbuf, tbuf2, crbuf, crbuf2, obuf,
                 sem, sem2):
    w = lax.axis_index("s") * NC + lax.axis_index("c")
    base = w * PER_W
    iota = lax.iota(jnp.int32, 16)

    for c in range(NCHUNK):
        off = pl.multiple_of(base + c * CH, CH)
        pltpu.sync_copy(rel_hbm.at[pl.ds(off, CH)], relbuf.at[c])

    hbufs = (hbuf, hbuf2)
    tbufs = (tbuf, tbuf2)
    crbufs = (crbuf, crbuf2)
    sems = (sem, sem2)

    def fire(c):
        off = pl.multiple_of(base + c * CH, CH)
        s = c % 2
        return [
            pltpu.async_copy(rows_hbm.at[pl.ds(off, CH), :], hbufs[s],
                             sems[s]),
            pltpu.async_copy(rows_hbm.at[pl.ds(B + off, CH), :], tbufs[s],
                             sems[s]),
            pltpu.async_copy(cr_hbm.at[relbuf.at[c]], crbufs[s], sems[s]),
        ]

    pending = {0: fire(0)}
    for c in range(NCHUNK):
        if c + 1 < NCHUNK:
            pending[c + 1] = fire(c + 1)
        for cp in pending.pop(c):
            cp.wait()
        hbuf_c, tbuf_c, crbuf_c = hbufs[c % 2], tbufs[c % 2], crbufs[c % 2]

        def row(i, carry):
            s = jnp.zeros((16,), jnp.float32)
            for q in range(D // 16):
                qo = pl.multiple_of(q * 16, 16)
                qo2 = pl.multiple_of(D + q * 16, 16)
                h = hbuf_c[i, pl.ds(qo, 16)]
                t = tbuf_c[i, pl.ds(qo, 16)]
                cc = crbuf_c[i, pl.ds(qo, 16)]
                rr = crbuf_c[i, pl.ds(qo2, 16)]
                diff = h * cc + rr - t
                s = s + diff * diff
            for sh in (1, 2, 4, 8):
                s = s + s.at[iota ^ sh].get(mode="promise_in_bounds")
            lane = i & jnp.int32(15)
            m = iota == lane
            idx16 = jnp.full((16,), c * CH, jnp.int32) + i
            plsc.store_scatter(obuf, [idx16], _sqrt16(s), mask=m)
            return carry

        lax.fori_loop(0, CH, row, 0)

    pltpu.sync_copy(obuf, out_hbm.at[pl.ds(pl.multiple_of(base, PER_W), PER_W)])


_phase2 = functools.partial(
    pl.kernel,
    out_type=jax.ShapeDtypeStruct((B,), jnp.float32),
    compiler_params=_PARAMS_LINEAR,
    mesh=_MESH,
    scratch_types=[
        pltpu.VMEM((NCHUNK, CH), jnp.int32),
        pltpu.VMEM((CH, 2 * D), jnp.float32),
        pltpu.VMEM((CH, 2 * D), jnp.float32),
        pltpu.VMEM((CH, 2 * D), jnp.float32),
        pltpu.VMEM((CH, 2 * D), jnp.float32),
        pltpu.VMEM((CH, 2 * D), jnp.float32),
        pltpu.VMEM((CH, 2 * D), jnp.float32),
        pltpu.VMEM((PER_W,), jnp.float32),
        pltpu.SemaphoreType.DMA,
        pltpu.SemaphoreType.DMA,
    ],
)(_phase2_body)


def kernel(entity_embedding, relation_embedding, head, relation, tail):
    cr = _build_cr_table(relation_embedding)
    ent_t = jnp.swapaxes(entity_embedding, 0, 1)
    tailtab = jnp.pad(
        jnp.swapaxes(entity_embedding[TAIL_LO:], 0, 1), ((0, 0), (0, 448)))
    ht = jnp.concatenate([head, tail])
    rows = _phase1(ent_t, tailtab, ht)
    return _phase2(rows, cr, relation)
